# Initial kernel scaffold; baseline (speedup 1.0000x reference)
import jax
import jax.numpy as jnp
from jax.experimental import pallas as pl


def kernel(x, edge_index, W_w, W_b, a_w, a_b):
    raise NotImplementedError("write your pallas kernel here")


# speedup vs baseline: 11.7532x; 11.7532x over previous
import functools

import jax
import jax.numpy as jnp
from jax import lax
from jax.experimental import pallas as pl
from jax.experimental.pallas import tpu as pltpu
from jax.experimental.pallas import tpu_sc as plsc

N_NODES = 10000
N_EDGES = 320000
D = 128

NUM_CORES = 2
NUM_SUBCORES = 16
NUM_WORKERS = NUM_CORES * NUM_SUBCORES

CHUNK = 128
CHUNKS_PER_WORKER = 80
EDGES_PER_WORKER = CHUNKS_PER_WORKER * CHUNK
E_PAD = EDGES_PER_WORKER * NUM_WORKERS

ACC_ROWS = 10240
N_DUMMY = ACC_ROWS - N_NODES
ZERO_ROWS = ACC_ROWS // NUM_SUBCORES
OUT_ROWS_PER_TILE = ACC_ROWS // NUM_SUBCORES


def _matmul_body(x_ref, w_ref, b_ref, h_ref):
    h_ref[...] = lax.dot_general(
        x_ref[...], w_ref[...], (((1,), (1,)), ((), ())),
        preferred_element_type=jnp.float32,
    ) + b_ref[...]


def _linear(x, W_w, W_b):
    return pl.pallas_call(
        _matmul_body,
        grid=(10,),
        in_specs=[
            pl.BlockSpec((1000, D), lambda i: (i, 0)),
            pl.BlockSpec((D, D), lambda i: (0, 0)),
            pl.BlockSpec((1, D), lambda i: (0, 0)),
        ],
        out_specs=pl.BlockSpec((1000, D), lambda i: (i, 0)),
        out_shape=jax.ShapeDtypeStruct((N_NODES, D), jnp.float32),
    )(x, W_w, W_b.reshape(1, D))


def _combine_body(p_ref, o_ref):
    o_ref[...] = p_ref[0] + p_ref[1]


def _combine(partials):
    return pl.pallas_call(
        _combine_body,
        grid=(10,),
        in_specs=[pl.BlockSpec((NUM_CORES, 1000, D), lambda i: (0, i, 0))],
        out_specs=pl.BlockSpec((1000, D), lambda i: (i, 0)),
        out_shape=jax.ShapeDtypeStruct((N_NODES, D), jnp.float32),
    )(partials)


@functools.partial(
    pl.kernel,
    mesh=plsc.VectorSubcoreMesh(core_axis_name="c", subcore_axis_name="s"),
    out_type=jax.ShapeDtypeStruct((NUM_CORES, ACC_ROWS, D), jnp.float32),
    scratch_types=[
        pltpu.VMEM((CHUNK, D), jnp.float32),
        pltpu.VMEM((CHUNKS_PER_WORKER, CHUNK), jnp.int32),
        pltpu.VMEM((CHUNKS_PER_WORKER, CHUNK), jnp.int32),
        pltpu.VMEM_SHARED((ACC_ROWS, D), jnp.float32),
        pltpu.SemaphoreType.DMA,
    ],
)
def _sc_segment_sum(h_hbm, col_hbm, row_hbm, out_hbm,
                    chunk_v, col_v, row_v, acc_sh, sem):
    cid = lax.axis_index("c")
    sid = lax.axis_index("s")
    wid = cid * NUM_SUBCORES + sid

    def _zrow(i, _):
        for c in range(D // 16):
            chunk_v[i, pl.ds(c * 16, 16)] = jnp.zeros((16,), jnp.float32)
        return 0
    lax.fori_loop(0, CHUNK, _zrow, 0)
    for r in range(ZERO_ROWS // CHUNK):
        pltpu.sync_copy(
            chunk_v, acc_sh.at[pl.ds(sid * ZERO_ROWS + r * CHUNK, CHUNK)])

    base = wid * CHUNKS_PER_WORKER
    pltpu.sync_copy(col_hbm.at[pl.ds(base, CHUNKS_PER_WORKER)], col_v)
    pltpu.sync_copy(row_hbm.at[pl.ds(base, CHUNKS_PER_WORKER)], row_v)

    plsc.subcore_barrier()

    def _chunk(j, _):
        pltpu.async_copy(h_hbm.at[col_v.at[j]], chunk_v, sem).wait()
        pltpu.sync_copy(chunk_v, acc_sh.at[row_v.at[j]], add=True)
        return 0
    lax.fori_loop(0, CHUNKS_PER_WORKER, _chunk, 0)

    plsc.subcore_barrier()

    pltpu.sync_copy(
        acc_sh.at[pl.ds(sid * OUT_ROWS_PER_TILE, OUT_ROWS_PER_TILE)],
        out_hbm.at[cid, pl.ds(sid * OUT_ROWS_PER_TILE, OUT_ROWS_PER_TILE)],
    )


def kernel(x, edge_index, W_w, W_b, a_w, a_b):
    h = _linear(x, W_w, W_b)

    row = edge_index[0].astype(jnp.int32)
    col = edge_index[1].astype(jnp.int32)
    pad = E_PAD - N_EDGES
    ar = jnp.arange(pad, dtype=jnp.int32)
    pad_row = N_NODES + (ar % N_DUMMY)
    pad_col = (ar * 37) % N_NODES
    row2d = jnp.concatenate([row, pad_row]).reshape(E_PAD // CHUNK, CHUNK)
    col2d = jnp.concatenate([col, pad_col]).reshape(E_PAD // CHUNK, CHUNK)

    partials = _sc_segment_sum(h, col2d, row2d)
    return _combine(partials)


# speedup vs baseline: 14.8904x; 1.2669x over previous
import functools

import jax
import jax.numpy as jnp
from jax import lax
from jax.experimental import pallas as pl
from jax.experimental.pallas import tpu as pltpu
from jax.experimental.pallas import tpu_sc as plsc

N_NODES = 10000
N_EDGES = 320000
D = 128

NUM_CORES = 2
NUM_SUBCORES = 16
NUM_WORKERS = NUM_CORES * NUM_SUBCORES

CHUNK = 128
GRP = 8
CHUNKS_PER_WORKER = 80
EDGES_PER_WORKER = CHUNKS_PER_WORKER * CHUNK
E_PAD = EDGES_PER_WORKER * NUM_WORKERS

ACC_ROWS = 10240
N_DUMMY = ACC_ROWS - N_NODES
ZERO_ROWS = ACC_ROWS // NUM_SUBCORES
OUT_ROWS_PER_TILE = ACC_ROWS // NUM_SUBCORES


def _matmul_body(x_ref, w_ref, b_ref, h_ref):
    h_ref[...] = lax.dot_general(
        x_ref[...], w_ref[...], (((1,), (1,)), ((), ())),
        preferred_element_type=jnp.float32,
    ) + b_ref[...]


def _linear(x, W_w, W_b):
    return pl.pallas_call(
        _matmul_body,
        grid=(10,),
        in_specs=[
            pl.BlockSpec((1000, D), lambda i: (i, 0)),
            pl.BlockSpec((D, D), lambda i: (0, 0)),
            pl.BlockSpec((1, D), lambda i: (0, 0)),
        ],
        out_specs=pl.BlockSpec((1000, D), lambda i: (i, 0)),
        out_shape=jax.ShapeDtypeStruct((N_NODES, D), jnp.float32),
    )(x, W_w, W_b.reshape(1, D))


def _combine_body(p_ref, o_ref):
    o_ref[...] = p_ref[0] + p_ref[1]


def _combine(partials):
    return pl.pallas_call(
        _combine_body,
        grid=(10,),
        in_specs=[pl.BlockSpec((NUM_CORES, 1000, D), lambda i: (0, i, 0))],
        out_specs=pl.BlockSpec((1000, D), lambda i: (i, 0)),
        out_shape=jax.ShapeDtypeStruct((N_NODES, D), jnp.float32),
    )(partials)


@functools.partial(
    pl.kernel,
    mesh=plsc.VectorSubcoreMesh(core_axis_name="c", subcore_axis_name="s"),
    out_type=jax.ShapeDtypeStruct((NUM_CORES, ACC_ROWS, D), jnp.float32),
    scratch_types=[
        pltpu.VMEM((2, CHUNK, D), jnp.float32),
        pltpu.VMEM((GRP, CHUNK), jnp.int32),
        pltpu.VMEM((GRP, CHUNK), jnp.int32),
        pltpu.VMEM_SHARED((ACC_ROWS, D), jnp.float32),
        pltpu.SemaphoreType.DMA,
        pltpu.SemaphoreType.DMA,
    ],
)
def _sc_segment_sum(h_hbm, col_hbm, row_hbm, out_hbm,
                    buf_v, col_v, row_v, acc_sh, sem0, sem1):
    cid = lax.axis_index("c")
    sid = lax.axis_index("s")
    wid = cid * NUM_SUBCORES + sid
    sems = (sem0, sem1)

    zbuf = buf_v.at[0]

    def _zrow(i, _):
        for c in range(D // 16):
            zbuf[i, pl.ds(c * 16, 16)] = jnp.zeros((16,), jnp.float32)
        return 0
    lax.fori_loop(0, CHUNK, _zrow, 0)
    for r in range(ZERO_ROWS // CHUNK):
        pltpu.sync_copy(
            zbuf, acc_sh.at[pl.ds(sid * ZERO_ROWS + r * CHUNK, CHUNK)])

    plsc.subcore_barrier()

    base = wid * CHUNKS_PER_WORKER

    def _group(g, _):
        off = pl.multiple_of(base + g * GRP, 8)
        pltpu.sync_copy(col_hbm.at[pl.ds(off, GRP)], col_v)
        pltpu.sync_copy(row_hbm.at[pl.ds(off, GRP)], row_v)

        copies = [None, None]

        def _start(k):
            copies[k % 2] = pltpu.async_copy(
                h_hbm.at[col_v.at[k]], buf_v.at[k % 2], sems[k % 2])

        _start(0)
        for k in range(GRP):
            if k + 1 < GRP:
                _start(k + 1)
            copies[k % 2].wait()
            pltpu.sync_copy(buf_v.at[k % 2], acc_sh.at[row_v.at[k]], add=True)
        return 0
    lax.fori_loop(0, CHUNKS_PER_WORKER // GRP, _group, 0)

    plsc.subcore_barrier()

    pltpu.sync_copy(
        acc_sh.at[pl.ds(sid * OUT_ROWS_PER_TILE, OUT_ROWS_PER_TILE)],
        out_hbm.at[cid, pl.ds(sid * OUT_ROWS_PER_TILE, OUT_ROWS_PER_TILE)],
    )


def kernel(x, edge_index, W_w, W_b, a_w, a_b):
    h = _linear(x, W_w, W_b)

    row = edge_index[0].astype(jnp.int32)
    col = edge_index[1].astype(jnp.int32)
    pad = E_PAD - N_EDGES
    ar = jnp.arange(pad, dtype=jnp.int32)
    pad_row = N_NODES + (ar % N_DUMMY)
    pad_col = (ar * 37) % N_NODES
    row2d = jnp.concatenate([row, pad_row]).reshape(E_PAD // CHUNK, CHUNK)
    col2d = jnp.concatenate([col, pad_col]).reshape(E_PAD // CHUNK, CHUNK)

    partials = _sc_segment_sum(h, col2d, row2d)
    return _combine(partials)
